# Initial kernel scaffold; baseline (speedup 1.0000x reference)
import jax
import jax.numpy as jnp
from jax.experimental import pallas as pl


def kernel(x, edge_index, W1, att_src1, att_dst1, b1, W2, att_src2, att_dst2, b2):
    raise NotImplementedError("write your pallas kernel here")


# speedup vs baseline: 50.3815x; 50.3815x over previous
import jax
import jax.numpy as jnp
from jax import lax
from jax.experimental import pallas as pl
from jax.experimental.pallas import tpu as pltpu
from jax.experimental.pallas import tpu_sc as plsc

N = 10000
E = 320000
F_IN = 128
H = 8
D = 8
HD = H * D
HH = 4
HW = HH * D
C = 16

NC = 2
NS = 16
NW = NC * NS
EPW = E // NW
B = 80
CH = EPW // B
NPT = 640
NPT_LAST = N - (NS - 1) * NPT

G1W = 40
A1W = 40
G2W = 32

_f32 = jnp.float32
_i32 = jnp.int32

_SC_PARAMS = pltpu.CompilerParams(use_tc_tiling_on_sc=False,
                                  needs_layout_passes=False)


def _prep1_body(x_r, w_r, asel_r, dsel_r, g1a_r, g1b_r, ad1_r, mx_r):
    h = jnp.dot(x_r[...], w_r[...], preferred_element_type=_f32)
    asrc = jnp.dot(h, asel_r[...], preferred_element_type=_f32)
    adst = jnp.dot(h, dsel_r[...], preferred_element_type=_f32)
    rows = h.shape[0]
    zpad = jnp.zeros((rows, 4), _f32)
    g1a_r[...] = jnp.concatenate([h[:, 0:HW], asrc[:, 0:HH], zpad], axis=1)
    g1b_r[...] = jnp.concatenate([h[:, HW:HD], asrc[:, HH:H], zpad], axis=1)
    ad1_r[...] = adst
    m = jnp.concatenate([jnp.max(asrc, axis=0, keepdims=True),
                         jnp.max(adst, axis=0, keepdims=True)], axis=1)
    i = pl.program_id(0)

    @pl.when(i == 0)
    def _():
        mx_r[...] = m

    @pl.when(i > 0)
    def _():
        mx_r[...] = jnp.maximum(mx_r[...], m)


def _prep1(x, W1, asel, dsel):
    R = 2000
    grid = N // R
    return pl.pallas_call(
        _prep1_body,
        grid=(grid,),
        in_specs=[
            pl.BlockSpec((R, F_IN), lambda i: (i, 0)),
            pl.BlockSpec((F_IN, HD), lambda i: (0, 0)),
            pl.BlockSpec((HD, H), lambda i: (0, 0)),
            pl.BlockSpec((HD, H), lambda i: (0, 0)),
        ],
        out_specs=[
            pl.BlockSpec((R, G1W), lambda i: (i, 0)),
            pl.BlockSpec((R, G1W), lambda i: (i, 0)),
            pl.BlockSpec((R, H), lambda i: (i, 0)),
            pl.BlockSpec((1, 2 * H), lambda i: (0, 0)),
        ],
        out_shape=[
            jax.ShapeDtypeStruct((N, G1W), _f32),
            jax.ShapeDtypeStruct((N, G1W), _f32),
            jax.ShapeDtypeStruct((N, H), _f32),
            jax.ShapeDtypeStruct((1, 2 * H), _f32),
        ],
    )(x, W1, asel, dsel)


def _zero_acc(zb_hbm, acc, s):
    @pl.when(s < NS - 1)
    def _():
        pltpu.sync_copy(zb_hbm.at[pl.ds(s * NPT, NPT)],
                        acc.at[pl.ds(s * NPT, NPT)])

    @pl.when(s == NS - 1)
    def _():
        pltpu.sync_copy(zb_hbm.at[pl.ds(s * NPT, NPT_LAST)],
                        acc.at[pl.ds(s * NPT, NPT_LAST)])


def _write_acc(acc, out_hbm, c, s):
    @pl.when(s < NS - 1)
    def _():
        pltpu.sync_copy(acc.at[pl.ds(s * NPT, NPT)],
                        out_hbm.at[c, pl.ds(s * NPT, NPT)])

    @pl.when(s == NS - 1)
    def _():
        pltpu.sync_copy(acc.at[pl.ds(s * NPT, NPT_LAST)],
                        out_hbm.at[c, pl.ds(s * NPT, NPT_LAST)])


def _edge1_body(j0, g1_hbm, ad1_hbm, m_hbm, src_hbm, dst_hbm, zb_hbm, out_hbm,
                ga, outb, exb, sidx, didx, adt, mv, acc, sem):
    c = lax.axis_index("c")
    s = lax.axis_index("s")
    wid = c * NS + s

    pltpu.sync_copy(ad1_hbm, adt)
    pltpu.sync_copy(m_hbm, mv)
    _zero_acc(zb_hbm, acc, s)
    plsc.subcore_barrier()

    iota = lax.iota(_i32, 16)
    hi8 = (iota >= 8).astype(_i32)

    def chunk(k, carry):
        base = wid * EPW + k * B
        pltpu.sync_copy(src_hbm.at[pl.ds(base, B)], sidx)
        pltpu.sync_copy(dst_hbm.at[pl.ds(base, B)], didx)
        pltpu.async_copy(g1_hbm.at[sidx], ga, sem).wait()
        for i16 in range(B // 16):
            rows = iota + i16 * 16
            dvec = didx[pl.ds(i16 * 16, 16)]
            for j in range(HH):
                asr = plsc.load_gather(ga, [rows, jnp.full((16,), HW + j, _i32)])
                ads = plsc.load_gather(adt, [dvec, jnp.full((16,), j0 + j, _i32)])
                e = asr + ads
                e = jnp.where(e > 0, e, 0.2 * e)
                exj = jnp.exp(e - mv[j0 + j, :])
                plsc.store_scatter(exb, [rows, jnp.full((16,), j, _i32)], exj)
                plsc.store_scatter(outb, [rows, jnp.full((16,), HW + j, _i32)], exj)
        for i in range(B):
            exrow = exb[i, :]
            for q in range(2):
                idxq = hi8 + 2 * q
                exr = exrow.at[idxq].get(mode="promise_in_bounds")
                outb[i, pl.ds(q * 16, 16)] = ga[i, pl.ds(q * 16, 16)] * exr
        pltpu.sync_copy(outb, acc.at[didx], add=True)
        return carry

    lax.fori_loop(0, CH, chunk, 0)
    plsc.subcore_barrier()
    _write_acc(acc, out_hbm, c, s)


def _edge1(g1h, ad1, m16, src, dst, zb, j0):
    mesh = plsc.VectorSubcoreMesh(core_axis_name="c", subcore_axis_name="s",
                                  num_cores=NC, num_subcores=NS)
    f = pl.kernel(
        lambda *refs: _edge1_body(j0, *refs),
        out_type=jax.ShapeDtypeStruct((NC, N, A1W), _f32),
        mesh=mesh,
        compiler_params=_SC_PARAMS,
        scratch_types=[
            pltpu.VMEM((B, G1W), _f32),
            pltpu.VMEM((B, A1W), _f32),
            pltpu.VMEM((B, 16), _f32),
            pltpu.VMEM((B,), _i32),
            pltpu.VMEM((B,), _i32),
            pltpu.VMEM((N, H), _f32),
            pltpu.VMEM((H, 16), _f32),
            pltpu.VMEM_SHARED((N, A1W), _f32),
            pltpu.SemaphoreType.DMA,
        ],
    )
    return f(g1h, ad1, m16, src, dst, zb)


def _mid_body(pa0_r, pa1_r, pb0_r, pb1_r, g1a_r, g1b_r, ad1_r, m1_r, b1_r,
              w2_r, s8_r, a2s_r, a2d_r, g2_r, ad2_r, mx_r):
    rows = g1a_r.shape[0]
    h = jnp.concatenate([g1a_r[:, 0:HW], g1b_r[:, 0:HW]], axis=1)
    asrc = jnp.concatenate([g1a_r[:, HW:HW + HH], g1b_r[:, HW:HW + HH]],
                           axis=1)
    adst = ad1_r[...]
    es = asrc + adst
    es = jnp.where(es > 0, es, 0.2 * es)
    exs = jnp.exp(es - m1_r[...])
    s8 = s8_r[...]
    ex_rep = jnp.dot(exs, s8, preferred_element_type=_f32)
    msg = jnp.concatenate([pa0_r[:, 0:HW] + pa1_r[:, 0:HW],
                           pb0_r[:, 0:HW] + pb1_r[:, 0:HW]], axis=1)
    msg = msg + h * ex_rep
    den = jnp.concatenate([pa0_r[:, HW:HW + HH] + pa1_r[:, HW:HW + HH],
                           pb0_r[:, HW:HW + HH] + pb1_r[:, HW:HW + HH]],
                          axis=1)
    den = den + exs
    den_rep = jnp.dot(den, s8, preferred_element_type=_f32)
    h1 = jnp.maximum(msg / den_rep + b1_r[...], 0.0)
    h2 = jnp.dot(h1, w2_r[...], preferred_element_type=_f32)
    asrc2 = jnp.sum(h2 * a2s_r[...], axis=1, keepdims=True)
    adst2 = jnp.sum(h2 * a2d_r[...], axis=1, keepdims=True)
    g2_r[...] = jnp.concatenate(
        [h2, asrc2, jnp.zeros((rows, G2W - C - 1), _f32)], axis=1)
    ad2_r[...] = jnp.concatenate([adst2, jnp.zeros((rows, 7), _f32)], axis=1)
    m = jnp.concatenate(
        [jnp.max(asrc2, axis=0, keepdims=True),
         jnp.max(adst2, axis=0, keepdims=True),
         jnp.full((1, 14), -1e30, _f32)], axis=1)
    i = pl.program_id(0)

    @pl.when(i == 0)
    def _():
        mx_r[...] = m

    @pl.when(i > 0)
    def _():
        mx_r[...] = jnp.maximum(mx_r[...], m)


def _mid(pa0, pa1, pb0, pb1, g1a, g1b, ad1, m1, b1, W2, s8, a2s, a2d):
    R = 2000
    grid = N // R
    return pl.pallas_call(
        _mid_body,
        grid=(grid,),
        in_specs=[
            pl.BlockSpec((R, A1W), lambda i: (i, 0)),
            pl.BlockSpec((R, A1W), lambda i: (i, 0)),
            pl.BlockSpec((R, A1W), lambda i: (i, 0)),
            pl.BlockSpec((R, A1W), lambda i: (i, 0)),
            pl.BlockSpec((R, G1W), lambda i: (i, 0)),
            pl.BlockSpec((R, G1W), lambda i: (i, 0)),
            pl.BlockSpec((R, H), lambda i: (i, 0)),
            pl.BlockSpec((1, H), lambda i: (0, 0)),
            pl.BlockSpec((1, HD), lambda i: (0, 0)),
            pl.BlockSpec((HD, C), lambda i: (0, 0)),
            pl.BlockSpec((H, HD), lambda i: (0, 0)),
            pl.BlockSpec((1, C), lambda i: (0, 0)),
            pl.BlockSpec((1, C), lambda i: (0, 0)),
        ],
        out_specs=[
            pl.BlockSpec((R, G2W), lambda i: (i, 0)),
            pl.BlockSpec((R, H), lambda i: (i, 0)),
            pl.BlockSpec((1, 16), lambda i: (0, 0)),
        ],
        out_shape=[
            jax.ShapeDtypeStruct((N, G2W), _f32),
            jax.ShapeDtypeStruct((N, H), _f32),
            jax.ShapeDtypeStruct((1, 16), _f32),
        ],
    )(pa0, pa1, pb0, pb1, g1a, g1b, ad1, m1, b1, W2, s8, a2s, a2d)


def _edge2_body(g2_hbm, ad2_hbm, m_hbm, src_hbm, dst_hbm, zb_hbm, out_hbm,
                ga, outb, sidx, didx, adt, mv, acc, sem):
    c = lax.axis_index("c")
    s = lax.axis_index("s")
    wid = c * NS + s

    pltpu.sync_copy(ad2_hbm, adt)
    pltpu.sync_copy(m_hbm, mv)
    _zero_acc(zb_hbm, acc, s)
    plsc.subcore_barrier()

    iota = lax.iota(_i32, 16)

    def chunk(k, carry):
        base = wid * EPW + k * B
        pltpu.sync_copy(src_hbm.at[pl.ds(base, B)], sidx)
        pltpu.sync_copy(dst_hbm.at[pl.ds(base, B)], didx)
        pltpu.async_copy(g2_hbm.at[sidx], ga, sem).wait()
        for i16 in range(B // 16):
            rows = iota + i16 * 16
            dvec = didx[pl.ds(i16 * 16, 16)]
            asr = plsc.load_gather(ga, [rows, jnp.full((16,), C, _i32)])
            ads = plsc.load_gather(adt, [dvec, jnp.full((16,), 0, _i32)])
            e = asr + ads
            e = jnp.where(e > 0, e, 0.2 * e)
            ex16 = jnp.exp(e - mv[...])
            plsc.store_scatter(outb, [rows, jnp.full((16,), C, _i32)], ex16)
            for i in range(16):
                r = i16 * 16 + i
                exr = ex16.at[jnp.full((16,), i, _i32)].get(
                    mode="promise_in_bounds")
                outb[r, pl.ds(0, 16)] = ga[r, pl.ds(0, 16)] * exr
        pltpu.sync_copy(outb, acc.at[didx], add=True)
        return carry

    lax.fori_loop(0, CH, chunk, 0)
    plsc.subcore_barrier()
    _write_acc(acc, out_hbm, c, s)


def _edge2(g2, ad2, m16, src, dst, zb):
    mesh = plsc.VectorSubcoreMesh(core_axis_name="c", subcore_axis_name="s",
                                  num_cores=NC, num_subcores=NS)
    f = pl.kernel(
        _edge2_body,
        out_type=jax.ShapeDtypeStruct((NC, N, G2W), _f32),
        mesh=mesh,
        compiler_params=_SC_PARAMS,
        scratch_types=[
            pltpu.VMEM((B, G2W), _f32),
            pltpu.VMEM((B, G2W), _f32),
            pltpu.VMEM((B,), _i32),
            pltpu.VMEM((B,), _i32),
            pltpu.VMEM((N, H), _f32),
            pltpu.VMEM((16,), _f32),
            pltpu.VMEM_SHARED((N, G2W), _f32),
            pltpu.SemaphoreType.DMA,
        ],
    )
    return f(g2, ad2, m16, src, dst, zb)


def _final_body(aa_r, ab_r, g2_r, ad2_r, m2_r, b2_r, logsm_r, sm_r, cs_r):
    h2 = g2_r[:, 0:C]
    a2s = g2_r[:, C:C + 1]
    a2d = ad2_r[:, 0:1]
    es = a2s + a2d
    es = jnp.where(es > 0, es, 0.2 * es)
    exs = jnp.exp(es - m2_r[0:1, 0:1])
    num = aa_r[:, 0:C] + ab_r[:, 0:C] + h2 * exs
    den = aa_r[:, C:C + 1] + ab_r[:, C:C + 1] + exs
    xo = num / den + b2_r[...]
    mx = jnp.max(xo, axis=1, keepdims=True)
    exv = jnp.exp(xo - mx)
    sv = jnp.sum(exv, axis=1, keepdims=True)
    logsm_r[...] = xo - mx - jnp.log(sv)
    sm = exv / sv
    sm_r[...] = sm
    p1 = jnp.max(sm, axis=1, keepdims=True)
    is_top = (sm == p1).astype(_f32)
    cnt = jnp.sum(is_top, axis=1, keepdims=True)
    masked = jnp.where(sm == p1, -jnp.inf, sm)
    second = jnp.where(cnt >= 2.0, p1, jnp.max(masked, axis=1, keepdims=True))
    calib = 1.0 - p1 + second
    tot = jnp.sum(calib) * jnp.ones((1, 8), _f32)
    i = pl.program_id(0)

    @pl.when(i == 0)
    def _():
        cs_r[...] = tot

    @pl.when(i > 0)
    def _():
        cs_r[...] = cs_r[...] + tot


def _final(accA, accB, g2, ad2, m2, b2):
    R = 2000
    grid = N // R
    return pl.pallas_call(
        _final_body,
        grid=(grid,),
        in_specs=[
            pl.BlockSpec((R, G2W), lambda i: (i, 0)),
            pl.BlockSpec((R, G2W), lambda i: (i, 0)),
            pl.BlockSpec((R, G2W), lambda i: (i, 0)),
            pl.BlockSpec((R, H), lambda i: (i, 0)),
            pl.BlockSpec((1, 8), lambda i: (0, 0)),
            pl.BlockSpec((1, C), lambda i: (0, 0)),
        ],
        out_specs=[
            pl.BlockSpec((R, C), lambda i: (i, 0)),
            pl.BlockSpec((R, C), lambda i: (i, 0)),
            pl.BlockSpec((1, 8), lambda i: (0, 0)),
        ],
        out_shape=[
            jax.ShapeDtypeStruct((N, C), _f32),
            jax.ShapeDtypeStruct((N, C), _f32),
            jax.ShapeDtypeStruct((1, 8), _f32),
        ],
    )(accA, accB, g2, ad2, m2, b2)


def kernel(x, edge_index, W1, att_src1, att_dst1, b1, W2, att_src2, att_dst2, b2):
    src = edge_index[0]
    dst = edge_index[1]

    eye8 = jnp.eye(H, dtype=_f32)
    asel = (eye8[:, None, :] * att_src1[:, :, None]).reshape(HD, H)
    dsel = (eye8[:, None, :] * att_dst1[:, :, None]).reshape(HD, H)
    s8 = jnp.repeat(eye8, D, axis=1)

    g1a, g1b, ad1, mx1 = _prep1(x, W1, asel, dsel)
    m1 = mx1[0, :H] + mx1[0, H:]
    m1 = jnp.where(m1 > 0, m1, 0.2 * m1)
    m1t = jnp.tile(m1[:, None], (1, 16))

    zb40 = jnp.zeros((N, A1W), _f32)
    pa = _edge1(g1a, ad1, m1t, src, dst, zb40, 0)
    pb = _edge1(g1b, ad1, m1t, src, dst, zb40, HH)

    m1r = m1[None, :]
    g2, ad2, mx2 = _mid(pa[0], pa[1], pb[0], pb[1], g1a, g1b, ad1, m1r,
                        b1[None, :], W2, s8, att_src2, att_dst2)
    m2 = mx2[0, 0] + mx2[0, 1]
    m2 = jnp.where(m2 > 0, m2, 0.2 * m2)
    m2t = jnp.full((16,), m2, _f32)

    zb32 = jnp.zeros((N, G2W), _f32)
    acc2 = _edge2(g2, ad2, m2t, src, dst, zb32)

    m2r = jnp.full((1, 8), m2, _f32)
    logsm, sm, cs = _final(acc2[0], acc2[1], g2, ad2, m2r, b2[None, :])
    calib_mean = cs[0, 0] / _f32(N)
    return (logsm, calib_mean, sm)


# speedup vs baseline: 82.8669x; 1.6448x over previous
import jax
import jax.numpy as jnp
from jax import lax
from jax.experimental import pallas as pl
from jax.experimental.pallas import tpu as pltpu
from jax.experimental.pallas import tpu_sc as plsc

N = 10000
E = 320000
F_IN = 128
H = 8
D = 8
HD = H * D
HH = 4
HW = HH * D
C = 16

NC = 2
NS = 16
NW = NC * NS
EPW = E // NW
B = 80
CH = EPW // B
EPS = E // NS
CH2 = EPS // B
PH1 = 50
NPT = 640
NPT_LAST = N - (NS - 1) * NPT

G1W = 40
A1W = 40
G2W = 32

_f32 = jnp.float32
_i32 = jnp.int32

_SC_PARAMS = pltpu.CompilerParams(use_tc_tiling_on_sc=False,
                                  needs_layout_passes=False)


def _prep1_body(x_r, w_r, asel_r, dsel_r, g1a_r, g1b_r, ad1a_r, ad1b_r,
                ad1_r, mx_r):
    h = jnp.dot(x_r[...], w_r[...], preferred_element_type=_f32)
    asrc = jnp.dot(h, asel_r[...], preferred_element_type=_f32)
    adst = jnp.dot(h, dsel_r[...], preferred_element_type=_f32)
    rows = h.shape[0]
    zpad = jnp.zeros((rows, 4), _f32)
    g1a_r[...] = jnp.concatenate([h[:, 0:HW], asrc[:, 0:HH], zpad], axis=1)
    g1b_r[...] = jnp.concatenate([h[:, HW:HD], asrc[:, HH:H], zpad], axis=1)
    ad1a_r[...] = adst[:, 0:HH]
    ad1b_r[...] = adst[:, HH:H]
    ad1_r[...] = adst
    m = jnp.concatenate([jnp.max(asrc, axis=0, keepdims=True),
                         jnp.max(adst, axis=0, keepdims=True)], axis=1)
    i = pl.program_id(0)

    @pl.when(i == 0)
    def _():
        mx_r[...] = m

    @pl.when(i > 0)
    def _():
        mx_r[...] = jnp.maximum(mx_r[...], m)


def _prep1(x, W1, asel, dsel):
    R = 2000
    grid = N // R
    return pl.pallas_call(
        _prep1_body,
        grid=(grid,),
        in_specs=[
            pl.BlockSpec((R, F_IN), lambda i: (i, 0)),
            pl.BlockSpec((F_IN, HD), lambda i: (0, 0)),
            pl.BlockSpec((HD, H), lambda i: (0, 0)),
            pl.BlockSpec((HD, H), lambda i: (0, 0)),
        ],
        out_specs=[
            pl.BlockSpec((R, G1W), lambda i: (i, 0)),
            pl.BlockSpec((R, G1W), lambda i: (i, 0)),
            pl.BlockSpec((R, HH), lambda i: (i, 0)),
            pl.BlockSpec((R, HH), lambda i: (i, 0)),
            pl.BlockSpec((R, H), lambda i: (i, 0)),
            pl.BlockSpec((1, 2 * H), lambda i: (0, 0)),
        ],
        out_shape=[
            jax.ShapeDtypeStruct((N, G1W), _f32),
            jax.ShapeDtypeStruct((N, G1W), _f32),
            jax.ShapeDtypeStruct((N, HH), _f32),
            jax.ShapeDtypeStruct((N, HH), _f32),
            jax.ShapeDtypeStruct((N, H), _f32),
            jax.ShapeDtypeStruct((1, 2 * H), _f32),
        ],
    )(x, W1, asel, dsel)


def _zero_acc(zb_hbm, acc, s):
    @pl.when(s < NS - 1)
    def _():
        pltpu.sync_copy(zb_hbm.at[pl.ds(s * NPT, NPT)],
                        acc.at[pl.ds(s * NPT, NPT)])

    @pl.when(s == NS - 1)
    def _():
        pltpu.sync_copy(zb_hbm.at[pl.ds(s * NPT, NPT_LAST)],
                        acc.at[pl.ds(s * NPT, NPT_LAST)])


def _write_acc(acc, out_hbm, c, s):
    @pl.when(s < NS - 1)
    def _():
        pltpu.sync_copy(acc.at[pl.ds(s * NPT, NPT)],
                        out_hbm.at[c, pl.ds(s * NPT, NPT)])

    @pl.when(s == NS - 1)
    def _():
        pltpu.sync_copy(acc.at[pl.ds(s * NPT, NPT_LAST)],
                        out_hbm.at[c, pl.ds(s * NPT, NPT_LAST)])


def _edge1_compute(gap, outb, exb, didxs, adt, mv, k, j0, iota, hi8):
    for i16 in range(B // 16):
        rows = iota + i16 * 16
        dvec = didxs[k, pl.ds(i16 * 16, 16)]
        for j in range(HH):
            asr = plsc.load_gather(gap, [rows, jnp.full((16,), HW + j, _i32)])
            ads = plsc.load_gather(adt, [dvec, jnp.full((16,), j, _i32)])
            e = asr + ads
            e = jnp.where(e > 0, e, 0.2 * e)
            exj = jnp.exp(e - mv[j0 + j, :])
            plsc.store_scatter(exb, [rows, jnp.full((16,), j, _i32)], exj)
            plsc.store_scatter(outb, [rows, jnp.full((16,), HW + j, _i32)], exj)
    for i in range(B):
        exrow = exb[i, :]
        for q in range(2):
            idxq = hi8 + 2 * q
            exr = exrow.at[idxq].get(mode="promise_in_bounds")
            outb[i, pl.ds(q * 16, 16)] = gap[i, pl.ds(q * 16, 16)] * exr


def _edge1_body(g1ab_hbm, ad1ab_hbm, m_hbm, src2_hbm, dst2_hbm, zb_hbm,
                out_hbm, ga, outb, exb, sidxs, didxs, adt, mv, acc, sem):
    c = lax.axis_index("c")
    s = lax.axis_index("s")
    j0 = c * HH

    pltpu.sync_copy(ad1ab_hbm.at[c], adt)
    pltpu.sync_copy(m_hbm, mv)
    _zero_acc(zb_hbm, acc, s)
    plsc.subcore_barrier()

    iota = lax.iota(_i32, 16)
    hi8 = (iota >= 8).astype(_i32)
    g_hbm = g1ab_hbm.at[c]

    def phase(ph, carry):
        base = s * CH2 + ph * PH1
        pltpu.sync_copy(src2_hbm.at[pl.ds(base, PH1)], sidxs)
        pltpu.sync_copy(dst2_hbm.at[pl.ds(base, PH1)], didxs)
        pltpu.async_copy(g_hbm.at[sidxs.at[0]], ga.at[0], sem.at[0])

        def chunk(k, c2):
            pp = lax.rem(k, 2)
            pn = 1 - pp
            pltpu.make_async_copy(g_hbm.at[sidxs.at[k]], ga.at[pp],
                                  sem.at[pp]).wait()

            @pl.when(k < PH1 - 1)
            def _():
                pltpu.async_copy(g_hbm.at[sidxs.at[k + 1]], ga.at[pn],
                                 sem.at[pn])

            _edge1_compute(ga.at[pp], outb, exb, didxs, adt, mv, k, j0,
                           iota, hi8)
            pltpu.sync_copy(outb, acc.at[didxs.at[k]], add=True)
            return c2

        lax.fori_loop(0, PH1, chunk, 0)
        return carry

    lax.fori_loop(0, CH2 // PH1, phase, 0)
    plsc.subcore_barrier()
    _write_acc(acc, out_hbm, c, s)


def _edge1(g1ab, ad1ab, m16, src2, dst2, zb):
    mesh = plsc.VectorSubcoreMesh(core_axis_name="c", subcore_axis_name="s",
                                  num_cores=NC, num_subcores=NS)
    f = pl.kernel(
        _edge1_body,
        out_type=jax.ShapeDtypeStruct((NC, N, A1W), _f32),
        mesh=mesh,
        compiler_params=_SC_PARAMS,
        scratch_types=[
            pltpu.VMEM((2, B, G1W), _f32),
            pltpu.VMEM((B, A1W), _f32),
            pltpu.VMEM((B, 16), _f32),
            pltpu.VMEM((PH1, B), _i32),
            pltpu.VMEM((PH1, B), _i32),
            pltpu.VMEM((N, HH), _f32),
            pltpu.VMEM((H, 16), _f32),
            pltpu.VMEM_SHARED((N, A1W), _f32),
            pltpu.SemaphoreType.DMA((2,)),
        ],
    )
    return f(g1ab, ad1ab, m16, src2, dst2, zb)


def _mid_body(pa_r, pb_r, g1a_r, g1b_r, ad1_r, m1_r, b1_r,
              w2_r, s8_r, a2s_r, a2d_r, g2_r, ad2_r, mx_r):
    rows = g1a_r.shape[0]
    h = jnp.concatenate([g1a_r[:, 0:HW], g1b_r[:, 0:HW]], axis=1)
    asrc = jnp.concatenate([g1a_r[:, HW:HW + HH], g1b_r[:, HW:HW + HH]],
                           axis=1)
    adst = ad1_r[...]
    es = asrc + adst
    es = jnp.where(es > 0, es, 0.2 * es)
    exs = jnp.exp(es - m1_r[...])
    s8 = s8_r[...]
    ex_rep = jnp.dot(exs, s8, preferred_element_type=_f32)
    msg = jnp.concatenate([pa_r[:, 0:HW], pb_r[:, 0:HW]], axis=1)
    msg = msg + h * ex_rep
    den = jnp.concatenate([pa_r[:, HW:HW + HH], pb_r[:, HW:HW + HH]], axis=1)
    den = den + exs
    den_rep = jnp.dot(den, s8, preferred_element_type=_f32)
    h1 = jnp.maximum(msg / den_rep + b1_r[...], 0.0)
    h2 = jnp.dot(h1, w2_r[...], preferred_element_type=_f32)
    asrc2 = jnp.sum(h2 * a2s_r[...], axis=1, keepdims=True)
    adst2 = jnp.sum(h2 * a2d_r[...], axis=1, keepdims=True)
    g2_r[...] = jnp.concatenate(
        [h2, asrc2, jnp.zeros((rows, G2W - C - 1), _f32)], axis=1)
    ad2_r[...] = jnp.concatenate([adst2, jnp.zeros((rows, 7), _f32)], axis=1)
    m = jnp.concatenate(
        [jnp.max(asrc2, axis=0, keepdims=True),
         jnp.max(adst2, axis=0, keepdims=True),
         jnp.full((1, 14), -1e30, _f32)], axis=1)
    i = pl.program_id(0)

    @pl.when(i == 0)
    def _():
        mx_r[...] = m

    @pl.when(i > 0)
    def _():
        mx_r[...] = jnp.maximum(mx_r[...], m)


def _mid(pa, pb, g1a, g1b, ad1, m1, b1, W2, s8, a2s, a2d):
    R = 2000
    grid = N // R
    return pl.pallas_call(
        _mid_body,
        grid=(grid,),
        in_specs=[
            pl.BlockSpec((R, A1W), lambda i: (i, 0)),
            pl.BlockSpec((R, A1W), lambda i: (i, 0)),
            pl.BlockSpec((R, G1W), lambda i: (i, 0)),
            pl.BlockSpec((R, G1W), lambda i: (i, 0)),
            pl.BlockSpec((R, H), lambda i: (i, 0)),
            pl.BlockSpec((1, H), lambda i: (0, 0)),
            pl.BlockSpec((1, HD), lambda i: (0, 0)),
            pl.BlockSpec((HD, C), lambda i: (0, 0)),
            pl.BlockSpec((H, HD), lambda i: (0, 0)),
            pl.BlockSpec((1, C), lambda i: (0, 0)),
            pl.BlockSpec((1, C), lambda i: (0, 0)),
        ],
        out_specs=[
            pl.BlockSpec((R, G2W), lambda i: (i, 0)),
            pl.BlockSpec((R, H), lambda i: (i, 0)),
            pl.BlockSpec((1, 16), lambda i: (0, 0)),
        ],
        out_shape=[
            jax.ShapeDtypeStruct((N, G2W), _f32),
            jax.ShapeDtypeStruct((N, H), _f32),
            jax.ShapeDtypeStruct((1, 16), _f32),
        ],
    )(pa, pb, g1a, g1b, ad1, m1, b1, W2, s8, a2s, a2d)


def _edge2_compute(gap, outb, adt, mv, didxs, k, iota):
    for i16 in range(B // 16):
        rows = iota + i16 * 16
        dvec = didxs[k, pl.ds(i16 * 16, 16)]
        asr = plsc.load_gather(gap, [rows, jnp.full((16,), C, _i32)])
        ads = plsc.load_gather(adt, [dvec >> 4, dvec & 15])
        e = asr + ads
        e = jnp.where(e > 0, e, 0.2 * e)
        ex16 = jnp.exp(e - mv[...])
        plsc.store_scatter(outb, [rows, jnp.full((16,), C, _i32)], ex16)
        for i in range(16):
            r = i16 * 16 + i
            exr = ex16.at[jnp.full((16,), i, _i32)].get(
                mode="promise_in_bounds")
            outb[r, pl.ds(0, 16)] = gap[r, pl.ds(0, 16)] * exr


def _edge2_body(g2_hbm, ad2f_hbm, m_hbm, src2_hbm, dst2_hbm, zb_hbm, out_hbm,
                ga, outb, sidxs, didxs, adt, mv, acc, sem):
    c = lax.axis_index("c")
    s = lax.axis_index("s")
    wid = c * NS + s

    pltpu.sync_copy(src2_hbm.at[pl.ds(wid * CH, CH)], sidxs)
    pltpu.sync_copy(dst2_hbm.at[pl.ds(wid * CH, CH)], didxs)
    pltpu.sync_copy(ad2f_hbm, adt)
    pltpu.sync_copy(m_hbm, mv)
    _zero_acc(zb_hbm, acc, s)
    plsc.subcore_barrier()

    iota = lax.iota(_i32, 16)

    pltpu.async_copy(g2_hbm.at[sidxs.at[0]], ga.at[0], sem.at[0])

    def chunk(k, carry):
        pp = lax.rem(k, 2)
        pn = 1 - pp
        pltpu.make_async_copy(g2_hbm.at[sidxs.at[k]], ga.at[pp],
                              sem.at[pp]).wait()

        @pl.when(k < CH - 1)
        def _():
            pltpu.async_copy(g2_hbm.at[sidxs.at[k + 1]], ga.at[pn], sem.at[pn])

        _edge2_compute(ga.at[pp], outb, adt, mv, didxs, k, iota)
        pltpu.sync_copy(outb, acc.at[didxs.at[k]], add=True)
        return carry

    lax.fori_loop(0, CH, chunk, 0)
    plsc.subcore_barrier()
    _write_acc(acc, out_hbm, c, s)


def _edge2(g2, ad2f, m16, src2, dst2, zb):
    mesh = plsc.VectorSubcoreMesh(core_axis_name="c", subcore_axis_name="s",
                                  num_cores=NC, num_subcores=NS)
    f = pl.kernel(
        _edge2_body,
        out_type=jax.ShapeDtypeStruct((NC, N, G2W), _f32),
        mesh=mesh,
        compiler_params=_SC_PARAMS,
        scratch_types=[
            pltpu.VMEM((2, B, G2W), _f32),
            pltpu.VMEM((B, G2W), _f32),
            pltpu.VMEM((CH, B), _i32),
            pltpu.VMEM((CH, B), _i32),
            pltpu.VMEM((N // 16, 16), _f32),
            pltpu.VMEM((16,), _f32),
            pltpu.VMEM_SHARED((N, G2W), _f32),
            pltpu.SemaphoreType.DMA((2,)),
        ],
    )
    return f(g2, ad2f, m16, src2, dst2, zb)


def _final_body(aa_r, ab_r, g2_r, ad2_r, m2_r, b2_r, logsm_r, sm_r, cs_r):
    h2 = g2_r[:, 0:C]
    a2s = g2_r[:, C:C + 1]
    a2d = ad2_r[:, 0:1]
    es = a2s + a2d
    es = jnp.where(es > 0, es, 0.2 * es)
    exs = jnp.exp(es - m2_r[0:1, 0:1])
    num = aa_r[:, 0:C] + ab_r[:, 0:C] + h2 * exs
    den = aa_r[:, C:C + 1] + ab_r[:, C:C + 1] + exs
    xo = num / den + b2_r[...]
    mx = jnp.max(xo, axis=1, keepdims=True)
    exv = jnp.exp(xo - mx)
    sv = jnp.sum(exv, axis=1, keepdims=True)
    logsm_r[...] = xo - mx - jnp.log(sv)
    sm = exv / sv
    sm_r[...] = sm
    p1 = jnp.max(sm, axis=1, keepdims=True)
    is_top = (sm == p1).astype(_f32)
    cnt = jnp.sum(is_top, axis=1, keepdims=True)
    masked = jnp.where(sm == p1, -jnp.inf, sm)
    second = jnp.where(cnt >= 2.0, p1, jnp.max(masked, axis=1, keepdims=True))
    calib = 1.0 - p1 + second
    tot = jnp.sum(calib) * jnp.ones((1, 8), _f32)
    i = pl.program_id(0)

    @pl.when(i == 0)
    def _():
        cs_r[...] = tot

    @pl.when(i > 0)
    def _():
        cs_r[...] = cs_r[...] + tot


def _final(accA, accB, g2, ad2, m2, b2):
    R = 2000
    grid = N // R
    return pl.pallas_call(
        _final_body,
        grid=(grid,),
        in_specs=[
            pl.BlockSpec((R, G2W), lambda i: (i, 0)),
            pl.BlockSpec((R, G2W), lambda i: (i, 0)),
            pl.BlockSpec((R, G2W), lambda i: (i, 0)),
            pl.BlockSpec((R, H), lambda i: (i, 0)),
            pl.BlockSpec((1, 8), lambda i: (0, 0)),
            pl.BlockSpec((1, C), lambda i: (0, 0)),
        ],
        out_specs=[
            pl.BlockSpec((R, C), lambda i: (i, 0)),
            pl.BlockSpec((R, C), lambda i: (i, 0)),
            pl.BlockSpec((1, 8), lambda i: (0, 0)),
        ],
        out_shape=[
            jax.ShapeDtypeStruct((N, C), _f32),
            jax.ShapeDtypeStruct((N, C), _f32),
            jax.ShapeDtypeStruct((1, 8), _f32),
        ],
    )(accA, accB, g2, ad2, m2, b2)


def kernel(x, edge_index, W1, att_src1, att_dst1, b1, W2, att_src2, att_dst2, b2):
    src = edge_index[0]
    dst = edge_index[1]

    eye8 = jnp.eye(H, dtype=_f32)
    asel = (eye8[:, None, :] * att_src1[:, :, None]).reshape(HD, H)
    dsel = (eye8[:, None, :] * att_dst1[:, :, None]).reshape(HD, H)
    s8 = jnp.repeat(eye8, D, axis=1)

    src2 = src.reshape(E // B, B)
    dst2 = dst.reshape(E // B, B)

    g1a, g1b, ad1a, ad1b, ad1, mx1 = _prep1(x, W1, asel, dsel)
    m1 = mx1[0, :H] + mx1[0, H:]
    m1 = jnp.where(m1 > 0, m1, 0.2 * m1)
    m1t = jnp.tile(m1[:, None], (1, 16))

    g1ab = jnp.stack([g1a, g1b])
    ad1ab = jnp.stack([ad1a, ad1b])
    zb40 = jnp.zeros((N, A1W), _f32)
    p1 = _edge1(g1ab, ad1ab, m1t, src2, dst2, zb40)

    m1r = m1[None, :]
    g2, ad2, mx2 = _mid(p1[0], p1[1], g1a, g1b, ad1, m1r,
                        b1[None, :], W2, s8, att_src2, att_dst2)
    m2 = mx2[0, 0] + mx2[0, 1]
    m2 = jnp.where(m2 > 0, m2, 0.2 * m2)
    m2t = jnp.full((16,), m2, _f32)

    ad2f = ad2[:, 0].reshape(N // 16, 16)
    zb32 = jnp.zeros((N, G2W), _f32)
    acc2 = _edge2(g2, ad2f, m2t, src2, dst2, zb32)

    m2r = jnp.full((1, 8), m2, _f32)
    logsm, sm, cs = _final(acc2[0], acc2[1], g2, ad2, m2r, b2[None, :])
    calib_mean = cs[0, 0] / _f32(N)
    return (logsm, calib_mean, sm)


# speedup vs baseline: 98.2231x; 1.1853x over previous
import jax
import jax.numpy as jnp
from jax import lax
from jax.experimental import pallas as pl
from jax.experimental.pallas import tpu as pltpu
from jax.experimental.pallas import tpu_sc as plsc

N = 10000
E = 320000
F_IN = 128
H = 8
D = 8
HD = H * D
HH = 4
HW = HH * D
C = 16

NC = 2
NS = 16
NW = NC * NS
EPW = E // NW
B = 80
CH = EPW // B
EPS = E // NS
CH2 = EPS // B
PH1 = 50
NPT = 640
NPT_LAST = N - (NS - 1) * NPT

G1W = 40
A1W = 40
G2W = 32

_f32 = jnp.float32
_i32 = jnp.int32

_SC_PARAMS = pltpu.CompilerParams(use_tc_tiling_on_sc=False,
                                  needs_layout_passes=False)


def _prep1_body(x_r, w_r, asel_r, dsel_r, g1a_r, g1b_r, ad1a_r, ad1b_r,
                ad1_r, mx_r):
    h = jnp.dot(x_r[...], w_r[...], preferred_element_type=_f32)
    asrc = jnp.dot(h, asel_r[...], preferred_element_type=_f32)
    adst = jnp.dot(h, dsel_r[...], preferred_element_type=_f32)
    rows = h.shape[0]
    zpad = jnp.zeros((rows, 4), _f32)
    g1a_r[...] = jnp.concatenate([h[:, 0:HW], asrc[:, 0:HH], zpad], axis=1)
    g1b_r[...] = jnp.concatenate([h[:, HW:HD], asrc[:, HH:H], zpad], axis=1)
    ad1a_r[...] = adst[:, 0:HH]
    ad1b_r[...] = adst[:, HH:H]
    ad1_r[...] = adst
    m = jnp.concatenate([jnp.max(asrc, axis=0, keepdims=True),
                         jnp.max(adst, axis=0, keepdims=True)], axis=1)
    i = pl.program_id(0)

    @pl.when(i == 0)
    def _():
        mx_r[...] = m

    @pl.when(i > 0)
    def _():
        mx_r[...] = jnp.maximum(mx_r[...], m)


def _prep1(x, W1, asel, dsel):
    R = 2000
    grid = N // R
    return pl.pallas_call(
        _prep1_body,
        grid=(grid,),
        in_specs=[
            pl.BlockSpec((R, F_IN), lambda i: (i, 0)),
            pl.BlockSpec((F_IN, HD), lambda i: (0, 0)),
            pl.BlockSpec((HD, H), lambda i: (0, 0)),
            pl.BlockSpec((HD, H), lambda i: (0, 0)),
        ],
        out_specs=[
            pl.BlockSpec((R, G1W), lambda i: (i, 0)),
            pl.BlockSpec((R, G1W), lambda i: (i, 0)),
            pl.BlockSpec((R, HH), lambda i: (i, 0)),
            pl.BlockSpec((R, HH), lambda i: (i, 0)),
            pl.BlockSpec((R, H), lambda i: (i, 0)),
            pl.BlockSpec((1, 2 * H), lambda i: (0, 0)),
        ],
        out_shape=[
            jax.ShapeDtypeStruct((N, G1W), _f32),
            jax.ShapeDtypeStruct((N, G1W), _f32),
            jax.ShapeDtypeStruct((N, HH), _f32),
            jax.ShapeDtypeStruct((N, HH), _f32),
            jax.ShapeDtypeStruct((N, H), _f32),
            jax.ShapeDtypeStruct((1, 2 * H), _f32),
        ],
    )(x, W1, asel, dsel)


def _zero_acc(zb_hbm, acc, s):
    @pl.when(s < NS - 1)
    def _():
        pltpu.sync_copy(zb_hbm.at[pl.ds(s * NPT, NPT)],
                        acc.at[pl.ds(s * NPT, NPT)])

    @pl.when(s == NS - 1)
    def _():
        pltpu.sync_copy(zb_hbm.at[pl.ds(s * NPT, NPT_LAST)],
                        acc.at[pl.ds(s * NPT, NPT_LAST)])


def _write_acc(acc, out_hbm, c, s):
    @pl.when(s < NS - 1)
    def _():
        pltpu.sync_copy(acc.at[pl.ds(s * NPT, NPT)],
                        out_hbm.at[c, pl.ds(s * NPT, NPT)])

    @pl.when(s == NS - 1)
    def _():
        pltpu.sync_copy(acc.at[pl.ds(s * NPT, NPT_LAST)],
                        out_hbm.at[c, pl.ds(s * NPT, NPT_LAST)])


def _edge1_compute(gap, outb, exb, didxs, adt, mv, k, j0, iota, hi8):
    for i16 in range(B // 16):
        rows = iota + i16 * 16
        dvec = didxs[k, pl.ds(i16 * 16, 16)]
        for j in range(HH):
            asr = plsc.load_gather(gap, [rows, jnp.full((16,), HW + j, _i32)])
            ads = plsc.load_gather(adt, [dvec, jnp.full((16,), j, _i32)])
            e = asr + ads
            e = jnp.where(e > 0, e, 0.2 * e)
            exj = jnp.exp(e - mv[j0 + j, :])
            plsc.store_scatter(exb, [rows, jnp.full((16,), j, _i32)], exj)
            plsc.store_scatter(outb, [rows, jnp.full((16,), HW + j, _i32)], exj)
    for i in range(B):
        exrow = exb[i, :]
        for q in range(2):
            idxq = hi8 + 2 * q
            exr = exrow.at[idxq].get(mode="promise_in_bounds")
            outb[i, pl.ds(q * 16, 16)] = gap[i, pl.ds(q * 16, 16)] * exr


def _edge1_body(g1ab_hbm, ad1ab_hbm, m_hbm, src2_hbm, dst2_hbm, zb_hbm,
                out_hbm, ga, outb, exb, sidxs, didxs, adt, mv, acc, sem,
                sem2):
    c = lax.axis_index("c")
    s = lax.axis_index("s")
    j0 = c * HH

    pltpu.sync_copy(ad1ab_hbm.at[c], adt)
    pltpu.sync_copy(m_hbm, mv)
    _zero_acc(zb_hbm, acc, s)
    plsc.subcore_barrier()

    iota = lax.iota(_i32, 16)
    hi8 = (iota >= 8).astype(_i32)
    g_hbm = g1ab_hbm.at[c]

    def phase(ph, carry):
        base = s * CH2 + ph * PH1
        pltpu.sync_copy(src2_hbm.at[pl.ds(base, PH1)], sidxs)
        pltpu.sync_copy(dst2_hbm.at[pl.ds(base, PH1)], didxs)
        pltpu.async_copy(g_hbm.at[sidxs.at[0]], ga.at[0], sem.at[0])

        def chunk(k, c2):
            pp = lax.rem(k, 2)
            pn = 1 - pp
            pltpu.make_async_copy(g_hbm.at[sidxs.at[k]], ga.at[pp],
                                  sem.at[pp]).wait()

            @pl.when(k < PH1 - 1)
            def _():
                pltpu.async_copy(g_hbm.at[sidxs.at[k + 1]], ga.at[pn],
                                 sem.at[pn])

            @pl.when(k >= 2)
            def _():
                pltpu.make_async_copy(outb.at[pp], acc.at[didxs.at[k]],
                                      sem2.at[pp]).wait()

            _edge1_compute(ga.at[pp], outb.at[pp], exb, didxs, adt, mv, k,
                           j0, iota, hi8)
            pltpu.async_copy(outb.at[pp], acc.at[didxs.at[k]], sem2.at[pp],
                             add=True)
            return c2

        lax.fori_loop(0, PH1, chunk, 0)
        pltpu.make_async_copy(outb.at[0], acc.at[didxs.at[0]],
                              sem2.at[0]).wait()
        pltpu.make_async_copy(outb.at[1], acc.at[didxs.at[1]],
                              sem2.at[1]).wait()
        return carry

    lax.fori_loop(0, CH2 // PH1, phase, 0)
    plsc.subcore_barrier()
    _write_acc(acc, out_hbm, c, s)


def _edge1(g1ab, ad1ab, m16, src2, dst2, zb):
    mesh = plsc.VectorSubcoreMesh(core_axis_name="c", subcore_axis_name="s",
                                  num_cores=NC, num_subcores=NS)
    f = pl.kernel(
        _edge1_body,
        out_type=jax.ShapeDtypeStruct((NC, N, A1W), _f32),
        mesh=mesh,
        compiler_params=_SC_PARAMS,
        scratch_types=[
            pltpu.VMEM((2, B, G1W), _f32),
            pltpu.VMEM((2, B, A1W), _f32),
            pltpu.VMEM((B, 16), _f32),
            pltpu.VMEM((PH1, B), _i32),
            pltpu.VMEM((PH1, B), _i32),
            pltpu.VMEM((N, HH), _f32),
            pltpu.VMEM((H, 16), _f32),
            pltpu.VMEM_SHARED((N, A1W), _f32),
            pltpu.SemaphoreType.DMA((2,)),
            pltpu.SemaphoreType.DMA((2,)),
        ],
    )
    return f(g1ab, ad1ab, m16, src2, dst2, zb)


def _mid_body(pa_r, pb_r, g1a_r, g1b_r, ad1_r, m1_r, b1_r,
              w2_r, s8_r, a2s_r, a2d_r, g2_r, ad2_r, mx_r):
    rows = g1a_r.shape[0]
    h = jnp.concatenate([g1a_r[:, 0:HW], g1b_r[:, 0:HW]], axis=1)
    asrc = jnp.concatenate([g1a_r[:, HW:HW + HH], g1b_r[:, HW:HW + HH]],
                           axis=1)
    adst = ad1_r[...]
    es = asrc + adst
    es = jnp.where(es > 0, es, 0.2 * es)
    exs = jnp.exp(es - m1_r[...])
    s8 = s8_r[...]
    ex_rep = jnp.dot(exs, s8, preferred_element_type=_f32)
    msg = jnp.concatenate([pa_r[:, 0:HW], pb_r[:, 0:HW]], axis=1)
    msg = msg + h * ex_rep
    den = jnp.concatenate([pa_r[:, HW:HW + HH], pb_r[:, HW:HW + HH]], axis=1)
    den = den + exs
    den_rep = jnp.dot(den, s8, preferred_element_type=_f32)
    h1 = jnp.maximum(msg / den_rep + b1_r[...], 0.0)
    h2 = jnp.dot(h1, w2_r[...], preferred_element_type=_f32)
    asrc2 = jnp.sum(h2 * a2s_r[...], axis=1, keepdims=True)
    adst2 = jnp.sum(h2 * a2d_r[...], axis=1, keepdims=True)
    g2_r[...] = jnp.concatenate(
        [h2, asrc2, jnp.zeros((rows, G2W - C - 1), _f32)], axis=1)
    ad2_r[...] = jnp.concatenate([adst2, jnp.zeros((rows, 7), _f32)], axis=1)
    m = jnp.concatenate(
        [jnp.max(asrc2, axis=0, keepdims=True),
         jnp.max(adst2, axis=0, keepdims=True),
         jnp.full((1, 14), -1e30, _f32)], axis=1)
    i = pl.program_id(0)

    @pl.when(i == 0)
    def _():
        mx_r[...] = m

    @pl.when(i > 0)
    def _():
        mx_r[...] = jnp.maximum(mx_r[...], m)


def _mid(pa, pb, g1a, g1b, ad1, m1, b1, W2, s8, a2s, a2d):
    R = 2000
    grid = N // R
    return pl.pallas_call(
        _mid_body,
        grid=(grid,),
        in_specs=[
            pl.BlockSpec((R, A1W), lambda i: (i, 0)),
            pl.BlockSpec((R, A1W), lambda i: (i, 0)),
            pl.BlockSpec((R, G1W), lambda i: (i, 0)),
            pl.BlockSpec((R, G1W), lambda i: (i, 0)),
            pl.BlockSpec((R, H), lambda i: (i, 0)),
            pl.BlockSpec((1, H), lambda i: (0, 0)),
            pl.BlockSpec((1, HD), lambda i: (0, 0)),
            pl.BlockSpec((HD, C), lambda i: (0, 0)),
            pl.BlockSpec((H, HD), lambda i: (0, 0)),
            pl.BlockSpec((1, C), lambda i: (0, 0)),
            pl.BlockSpec((1, C), lambda i: (0, 0)),
        ],
        out_specs=[
            pl.BlockSpec((R, G2W), lambda i: (i, 0)),
            pl.BlockSpec((R, H), lambda i: (i, 0)),
            pl.BlockSpec((1, 16), lambda i: (0, 0)),
        ],
        out_shape=[
            jax.ShapeDtypeStruct((N, G2W), _f32),
            jax.ShapeDtypeStruct((N, H), _f32),
            jax.ShapeDtypeStruct((1, 16), _f32),
        ],
    )(pa, pb, g1a, g1b, ad1, m1, b1, W2, s8, a2s, a2d)


def _edge2_compute(gap, outb, adt, mv, didxs, k, iota):
    for i16 in range(B // 16):
        rows = iota + i16 * 16
        dvec = didxs[k, pl.ds(i16 * 16, 16)]
        asr = plsc.load_gather(gap, [rows, jnp.full((16,), C, _i32)])
        ads = plsc.load_gather(adt, [dvec >> 4, dvec & 15])
        e = asr + ads
        e = jnp.where(e > 0, e, 0.2 * e)
        ex16 = jnp.exp(e - mv[...])
        plsc.store_scatter(outb, [rows, jnp.full((16,), C, _i32)], ex16)
        for i in range(16):
            r = i16 * 16 + i
            exr = ex16.at[jnp.full((16,), i, _i32)].get(
                mode="promise_in_bounds")
            outb[r, pl.ds(0, 16)] = gap[r, pl.ds(0, 16)] * exr


def _edge2_body(g2_hbm, ad2f_hbm, m_hbm, src2_hbm, dst2_hbm, zb_hbm, out_hbm,
                ga, outb, sidxs, didxs, adt, mv, acc, sem, sem2):
    c = lax.axis_index("c")
    s = lax.axis_index("s")
    wid = c * NS + s

    pltpu.sync_copy(src2_hbm.at[pl.ds(wid * CH, CH)], sidxs)
    pltpu.sync_copy(dst2_hbm.at[pl.ds(wid * CH, CH)], didxs)
    pltpu.sync_copy(ad2f_hbm, adt)
    pltpu.sync_copy(m_hbm, mv)
    _zero_acc(zb_hbm, acc, s)
    plsc.subcore_barrier()

    iota = lax.iota(_i32, 16)

    pltpu.async_copy(g2_hbm.at[sidxs.at[0]], ga.at[0], sem.at[0])

    def chunk(k, carry):
        pp = lax.rem(k, 2)
        pn = 1 - pp
        pltpu.make_async_copy(g2_hbm.at[sidxs.at[k]], ga.at[pp],
                              sem.at[pp]).wait()

        @pl.when(k < CH - 1)
        def _():
            pltpu.async_copy(g2_hbm.at[sidxs.at[k + 1]], ga.at[pn], sem.at[pn])

        @pl.when(k >= 2)
        def _():
            pltpu.make_async_copy(outb.at[pp], acc.at[didxs.at[k]],
                                  sem2.at[pp]).wait()

        _edge2_compute(ga.at[pp], outb.at[pp], adt, mv, didxs, k, iota)
        pltpu.async_copy(outb.at[pp], acc.at[didxs.at[k]], sem2.at[pp],
                         add=True)
        return carry

    lax.fori_loop(0, CH, chunk, 0)
    pltpu.make_async_copy(outb.at[0], acc.at[didxs.at[0]], sem2.at[0]).wait()
    pltpu.make_async_copy(outb.at[1], acc.at[didxs.at[1]], sem2.at[1]).wait()
    plsc.subcore_barrier()
    _write_acc(acc, out_hbm, c, s)


def _edge2(g2, ad2f, m16, src2, dst2, zb):
    mesh = plsc.VectorSubcoreMesh(core_axis_name="c", subcore_axis_name="s",
                                  num_cores=NC, num_subcores=NS)
    f = pl.kernel(
        _edge2_body,
        out_type=jax.ShapeDtypeStruct((NC, N, G2W), _f32),
        mesh=mesh,
        compiler_params=_SC_PARAMS,
        scratch_types=[
            pltpu.VMEM((2, B, G2W), _f32),
            pltpu.VMEM((2, B, G2W), _f32),
            pltpu.VMEM((CH, B), _i32),
            pltpu.VMEM((CH, B), _i32),
            pltpu.VMEM((N // 16, 16), _f32),
            pltpu.VMEM((16,), _f32),
            pltpu.VMEM_SHARED((N, G2W), _f32),
            pltpu.SemaphoreType.DMA((2,)),
            pltpu.SemaphoreType.DMA((2,)),
        ],
    )
    return f(g2, ad2f, m16, src2, dst2, zb)


def _final_body(aa_r, ab_r, g2_r, ad2_r, m2_r, b2_r, logsm_r, sm_r, cs_r):
    h2 = g2_r[:, 0:C]
    a2s = g2_r[:, C:C + 1]
    a2d = ad2_r[:, 0:1]
    es = a2s + a2d
    es = jnp.where(es > 0, es, 0.2 * es)
    exs = jnp.exp(es - m2_r[0:1, 0:1])
    num = aa_r[:, 0:C] + ab_r[:, 0:C] + h2 * exs
    den = aa_r[:, C:C + 1] + ab_r[:, C:C + 1] + exs
    xo = num / den + b2_r[...]
    mx = jnp.max(xo, axis=1, keepdims=True)
    exv = jnp.exp(xo - mx)
    sv = jnp.sum(exv, axis=1, keepdims=True)
    logsm_r[...] = xo - mx - jnp.log(sv)
    sm = exv / sv
    sm_r[...] = sm
    p1 = jnp.max(sm, axis=1, keepdims=True)
    is_top = (sm == p1).astype(_f32)
    cnt = jnp.sum(is_top, axis=1, keepdims=True)
    masked = jnp.where(sm == p1, -jnp.inf, sm)
    second = jnp.where(cnt >= 2.0, p1, jnp.max(masked, axis=1, keepdims=True))
    calib = 1.0 - p1 + second
    tot = jnp.sum(calib) * jnp.ones((1, 8), _f32)
    i = pl.program_id(0)

    @pl.when(i == 0)
    def _():
        cs_r[...] = tot

    @pl.when(i > 0)
    def _():
        cs_r[...] = cs_r[...] + tot


def _final(accA, accB, g2, ad2, m2, b2):
    R = 2000
    grid = N // R
    return pl.pallas_call(
        _final_body,
        grid=(grid,),
        in_specs=[
            pl.BlockSpec((R, G2W), lambda i: (i, 0)),
            pl.BlockSpec((R, G2W), lambda i: (i, 0)),
            pl.BlockSpec((R, G2W), lambda i: (i, 0)),
            pl.BlockSpec((R, H), lambda i: (i, 0)),
            pl.BlockSpec((1, 8), lambda i: (0, 0)),
            pl.BlockSpec((1, C), lambda i: (0, 0)),
        ],
        out_specs=[
            pl.BlockSpec((R, C), lambda i: (i, 0)),
            pl.BlockSpec((R, C), lambda i: (i, 0)),
            pl.BlockSpec((1, 8), lambda i: (0, 0)),
        ],
        out_shape=[
            jax.ShapeDtypeStruct((N, C), _f32),
            jax.ShapeDtypeStruct((N, C), _f32),
            jax.ShapeDtypeStruct((1, 8), _f32),
        ],
    )(accA, accB, g2, ad2, m2, b2)


def kernel(x, edge_index, W1, att_src1, att_dst1, b1, W2, att_src2, att_dst2, b2):
    src = edge_index[0]
    dst = edge_index[1]

    eye8 = jnp.eye(H, dtype=_f32)
    asel = (eye8[:, None, :] * att_src1[:, :, None]).reshape(HD, H)
    dsel = (eye8[:, None, :] * att_dst1[:, :, None]).reshape(HD, H)
    s8 = jnp.repeat(eye8, D, axis=1)

    src2 = src.reshape(E // B, B)
    dst2 = dst.reshape(E // B, B)

    g1a, g1b, ad1a, ad1b, ad1, mx1 = _prep1(x, W1, asel, dsel)
    m1 = mx1[0, :H] + mx1[0, H:]
    m1 = jnp.where(m1 > 0, m1, 0.2 * m1)
    m1t = jnp.tile(m1[:, None], (1, 16))

    g1ab = jnp.stack([g1a, g1b])
    ad1ab = jnp.stack([ad1a, ad1b])
    zb40 = jnp.zeros((N, A1W), _f32)
    p1 = _edge1(g1ab, ad1ab, m1t, src2, dst2, zb40)

    m1r = m1[None, :]
    g2, ad2, mx2 = _mid(p1[0], p1[1], g1a, g1b, ad1, m1r,
                        b1[None, :], W2, s8, att_src2, att_dst2)
    m2 = mx2[0, 0] + mx2[0, 1]
    m2 = jnp.where(m2 > 0, m2, 0.2 * m2)
    m2t = jnp.full((16,), m2, _f32)

    ad2f = ad2[:, 0].reshape(N // 16, 16)
    zb32 = jnp.zeros((N, G2W), _f32)
    acc2 = _edge2(g2, ad2f, m2t, src2, dst2, zb32)

    m2r = jnp.full((1, 8), m2, _f32)
    logsm, sm, cs = _final(acc2[0], acc2[1], g2, ad2, m2r, b2[None, :])
    calib_mean = cs[0, 0] / _f32(N)
    return (logsm, calib_mean, sm)
